# baseline (speedup 1.0000x reference)
import functools

import jax
import jax.numpy as jnp
from jax import lax
from jax.experimental import pallas as pl
from jax.experimental.pallas import tpu as pltpu
from jax.experimental.pallas import tpu_sc as plsc

N = 10000
E = 320000
F_IN = 128
H = 64
C = 10
G = 64

NP = 10240
EP = 327680
EDGE_ROWS = EP // 128
HALF_ROWS = EDGE_ROWS // 2
TILE_EDGE_ROWS = EDGE_ROWS // 32
CHUNK_E = 1024
NCHUNK = TILE_EDGE_ROWS * 128 // CHUNK_E
NDBL_TOTAL = EP // (2 * CHUNK_E)
NDBL0 = 70
FB = 4
FG = H // FB

_mesh = functools.partial(
    plsc.VectorSubcoreMesh, core_axis_name="c", subcore_axis_name="s")
_sc_reg_params = pltpu.CompilerParams(
    use_tc_tiling_on_sc=False, needs_layout_passes=False)


@functools.partial(
    pl.kernel,
    out_type=jax.ShapeDtypeStruct((32, NP), jnp.float32),
    mesh=_mesh(),
    compiler_params=_sc_reg_params,
    scratch_types=[
        pltpu.VMEM((NP,), jnp.float32),
        pltpu.VMEM((CHUNK_E,), jnp.int32),
        pltpu.VMEM((CHUNK_E,), jnp.int32),
        pltpu.SemaphoreType.DMA,
        pltpu.SemaphoreType.DMA,
    ],
)
def _sc_degree(dst_hbm, out_hbm, dslab, didxa, didxb, sema, semb):
  cid = lax.axis_index("c")
  sid = lax.axis_index("s")
  tid = cid * 16 + sid
  base = tid * TILE_EDGE_ROWS * 128
  pltpu.async_copy(dst_hbm.at[pl.ds(base, CHUNK_E)], didxa, sema)
  zero16 = jnp.zeros((16,), jnp.float32)

  @plsc.parallel_loop(0, NP // 16, unroll=8)
  def _zero(i):
    dslab[pl.ds(i * 16, 16)] = zero16

  ones16 = jnp.ones((16,), jnp.float32)

  def process(didx):
    @plsc.parallel_loop(0, CHUNK_E // 16, unroll=8)
    def _groups(i):
      d = didx[pl.ds(i * 16, 16)]
      plsc.addupdate_scatter(dslab, [d], ones16)

  def body2(t, carry):
    r0 = base + t * 2 * CHUNK_E
    pltpu.async_copy(dst_hbm.at[pl.ds(r0 + CHUNK_E, CHUNK_E)], didxb, semb)
    pltpu.make_async_copy(dst_hbm.at[pl.ds(r0, CHUNK_E)], didxa, sema).wait()
    process(didxa)

    @pl.when(t < NCHUNK // 2 - 1)
    def _():
      pltpu.async_copy(dst_hbm.at[pl.ds(r0 + 2 * CHUNK_E, CHUNK_E)], didxa,
                       sema)

    pltpu.make_async_copy(dst_hbm.at[pl.ds(r0 + CHUNK_E, CHUNK_E)], didxb,
                          semb).wait()
    process(didxb)
    return carry

  lax.fori_loop(0, NCHUNK // 2, body2, 0)
  pltpu.sync_copy(dslab, out_hbm.at[tid])


@functools.partial(
    pl.kernel,
    out_type=jax.ShapeDtypeStruct((2, H, NP), jnp.float32),
    mesh=_mesh(),
    compiler_params=_sc_reg_params,
    scratch_types=[
        pltpu.VMEM((FB, NP), jnp.float32),
        pltpu.VMEM((FB, NP), jnp.float32),
        pltpu.VMEM((CHUNK_E,), jnp.int32),
        pltpu.VMEM((CHUNK_E,), jnp.int32),
        pltpu.VMEM((CHUNK_E,), jnp.int32),
        pltpu.VMEM((CHUNK_E,), jnp.int32),
        pltpu.SemaphoreType.DMA,
        pltpu.SemaphoreType.DMA,
    ],
)
def _sc_layer_reg(y_hbm, src_hbm, dst_hbm, out_hbm, yslab, zslab,
                  sidxa, didxa, sidxb, didxb, sema, semb):
  cid = lax.axis_index("c")
  sid = lax.axis_index("s")
  pltpu.async_copy(y_hbm.at[pl.ds(sid * FB, FB)], yslab, semb)
  zero16 = jnp.zeros((16,), jnp.float32)

  @plsc.parallel_loop(0, NP // 16, unroll=8)
  def _zero(i):
    for f in range(FB):
      zslab[f, pl.ds(i * 16, 16)] = zero16

  pltpu.make_async_copy(y_hbm.at[pl.ds(sid * FB, FB)], yslab, semb).wait()

  fsplat = [jnp.full((16,), f, jnp.int32) for f in range(FB)]

  def process(sidx, didx):
    @plsc.parallel_loop(0, CHUNK_E // 16, unroll=4)
    def _groups(i):
      s = sidx[pl.ds(i * 16, 16)]
      d = didx[pl.ds(i * 16, 16)]
      for f in range(FB):
        v = plsc.load_gather(yslab, [fsplat[f], s])
        plsc.addupdate_scatter(zslab, [fsplat[f], d], v)

  ndbl = jnp.where(cid == 0, NDBL0, NDBL_TOTAL - NDBL0)
  ebase = jnp.where(cid == 0, 0, NDBL0 * 2 * CHUNK_E)

  def wait_chunk(r, sidx, didx, sem):
    pltpu.make_async_copy(src_hbm.at[pl.ds(ebase + r, CHUNK_E)], sidx,
                          sem).wait()
    pltpu.make_async_copy(dst_hbm.at[pl.ds(ebase + r, CHUNK_E)], didx,
                          sem).wait()

  def load_chunk(r, sidx, didx, sem):
    pltpu.async_copy(src_hbm.at[pl.ds(ebase + r, CHUNK_E)], sidx, sem)
    pltpu.async_copy(dst_hbm.at[pl.ds(ebase + r, CHUNK_E)], didx, sem)

  load_chunk(0, sidxa, didxa, sema)

  def body2(t, carry):
    r0 = t * 2 * CHUNK_E
    load_chunk(r0 + CHUNK_E, sidxb, didxb, semb)
    wait_chunk(r0, sidxa, didxa, sema)
    process(sidxa, didxa)

    @pl.when(t < ndbl - 1)
    def _():
      load_chunk(r0 + 2 * CHUNK_E, sidxa, didxa, sema)

    wait_chunk(r0 + CHUNK_E, sidxb, didxb, semb)
    process(sidxb, didxb)
    return carry

  lax.fori_loop(0, ndbl, body2, 0)
  pltpu.sync_copy(zslab, out_hbm.at[cid, pl.ds(sid * FB, FB)])


def _tc_pre_body(deg_ref, xt_ref, w1t_ref, dinv_ref, y1_ref):
  deg = jnp.sum(deg_ref[...], axis=0, keepdims=True) + 1.0
  dinv = lax.rsqrt(deg)
  dinv_ref[...] = dinv
  xw = jnp.dot(w1t_ref[...], xt_ref[...], preferred_element_type=jnp.float32)
  y1_ref[...] = xw * dinv


def _tc_pre(deg2, x_fm, w1t):
  return pl.pallas_call(
      _tc_pre_body,
      out_shape=(jax.ShapeDtypeStruct((1, NP), jnp.float32),
                 jax.ShapeDtypeStruct((H, NP), jnp.float32)),
  )(deg2, x_fm, w1t)


def _tc_mid_body(z_ref, y_ref, dinv_ref, b_ref, wt_ref, ynext_ref):
  z = z_ref[0] + z_ref[1] + y_ref[...]
  dinv = dinv_ref[...]
  h = jnp.maximum(z * dinv + b_ref[...], 0.0)
  ynext_ref[...] = jnp.dot(
      wt_ref[...], h, preferred_element_type=jnp.float32) * dinv


def _tc_mid(z_parts, y_prev, dinv, b_col, wt_next):
  return pl.pallas_call(
      _tc_mid_body,
      out_shape=jax.ShapeDtypeStruct((H, NP), jnp.float32),
  )(z_parts, y_prev, dinv, b_col, wt_next)


def _tc_final_body(z_ref, y_ref, dinv_ref, b_ref, batch_ref, wl_ref, bl_ref,
                   out_ref, hpool_ref):
  z = z_ref[0] + z_ref[1] + y_ref[...]
  h = z * dinv_ref[...] + b_ref[...]
  h_n = h[:, :N]
  gids = lax.broadcasted_iota(jnp.int32, (N, G), 1)
  seg = (batch_ref[...] == gids).astype(jnp.float32)
  sums = jnp.dot(h_n, seg, preferred_element_type=jnp.float32)
  cnt = jnp.sum(seg, axis=0, keepdims=True)
  hpool_fm = sums / jnp.maximum(cnt, 1.0)
  eye = (lax.broadcasted_iota(jnp.int32, (H, H), 0) ==
         lax.broadcasted_iota(jnp.int32, (H, H), 1)).astype(jnp.float32)
  hpool_ref[...] = lax.dot_general(
      hpool_fm, eye, (((0,), (0,)), ((), ())),
      preferred_element_type=jnp.float32)
  out_ref[...] = lax.dot_general(
      hpool_fm, wl_ref[...], (((0,), (0,)), ((), ())),
      preferred_element_type=jnp.float32) + bl_ref[...]


def _tc_final(z_parts, y3, dinv, b3_col, batch_col, wl, bl):
  return pl.pallas_call(
      _tc_final_body,
      out_shape=(jax.ShapeDtypeStruct((G, C), jnp.float32),
                 jax.ShapeDtypeStruct((G, H), jnp.float32)),
  )(z_parts, y3, dinv, b3_col, batch_col, wl, bl)


def kernel(x, edge_index, batch, W1, b1, W2, b2, W3, b3, Wl, bl):
  src = edge_index[0]
  dst = edge_index[1]
  pad = jnp.full((EP - E,), N, dtype=jnp.int32)
  src1 = jnp.concatenate([src, pad])
  dst1 = jnp.concatenate([dst, pad])
  x_fm = jnp.pad(jnp.transpose(x), ((0, 0), (0, NP - N)))
  batch_col = batch.reshape(N, 1)

  deg_parts = _sc_degree(dst1)
  dinv, y1 = _tc_pre(deg_parts, x_fm, jnp.transpose(W1))

  z1 = _sc_layer_reg(y1, src1, dst1)
  y2 = _tc_mid(z1, y1, dinv, b1.reshape(H, 1), jnp.transpose(W2))

  z2 = _sc_layer_reg(y2, src1, dst1)
  y3 = _tc_mid(z2, y2, dinv, b2.reshape(H, 1), jnp.transpose(W3))

  z3 = _sc_layer_reg(y3, src1, dst1)
  out, hpool = _tc_final(z3, y3, dinv, b3.reshape(H, 1), batch_col,
                         Wl, bl.reshape(1, C))
  return (out, hpool)


# speedup vs baseline: 1.1597x; 1.1597x over previous
import functools

import jax
import jax.numpy as jnp
from jax import lax
from jax.experimental import pallas as pl
from jax.experimental.pallas import tpu as pltpu
from jax.experimental.pallas import tpu_sc as plsc

N = 10000
E = 320000
F_IN = 128
H = 64
C = 10
G = 64

NP = 10240
EP = 327680
EDGE_ROWS = EP // 128
HALF_ROWS = EDGE_ROWS // 2
TILE_EDGE_ROWS = EDGE_ROWS // 32
CHUNK_E = 1024
NCHUNK = TILE_EDGE_ROWS * 128 // CHUNK_E
NDBL_TOTAL = EP // (2 * CHUNK_E)
NDBL0 = 90
FB = 4
FG = H // FB

_mesh = functools.partial(
    plsc.VectorSubcoreMesh, core_axis_name="c", subcore_axis_name="s")
_sc_reg_params = pltpu.CompilerParams(
    use_tc_tiling_on_sc=False, needs_layout_passes=False)


@functools.partial(
    pl.kernel,
    out_type=jax.ShapeDtypeStruct((32, NP), jnp.float32),
    mesh=_mesh(),
    compiler_params=_sc_reg_params,
    scratch_types=[
        pltpu.VMEM((NP,), jnp.float32),
        pltpu.VMEM((CHUNK_E,), jnp.int32),
        pltpu.VMEM((CHUNK_E,), jnp.int32),
        pltpu.SemaphoreType.DMA,
        pltpu.SemaphoreType.DMA,
    ],
)
def _sc_degree(dst_hbm, out_hbm, dslab, didxa, didxb, sema, semb):
  cid = lax.axis_index("c")
  sid = lax.axis_index("s")
  tid = cid * 16 + sid
  base = tid * TILE_EDGE_ROWS * 128
  pltpu.async_copy(dst_hbm.at[pl.ds(base, CHUNK_E)], didxa, sema)
  zero16 = jnp.zeros((16,), jnp.float32)

  @plsc.parallel_loop(0, NP // 16, unroll=8)
  def _zero(i):
    dslab[pl.ds(i * 16, 16)] = zero16

  ones16 = jnp.ones((16,), jnp.float32)

  def process(didx):
    @plsc.parallel_loop(0, CHUNK_E // 16, unroll=8)
    def _groups(i):
      d = didx[pl.ds(i * 16, 16)]
      plsc.addupdate_scatter(dslab, [d], ones16)

  def body2(t, carry):
    r0 = base + t * 2 * CHUNK_E
    pltpu.async_copy(dst_hbm.at[pl.ds(r0 + CHUNK_E, CHUNK_E)], didxb, semb)
    pltpu.make_async_copy(dst_hbm.at[pl.ds(r0, CHUNK_E)], didxa, sema).wait()
    process(didxa)

    @pl.when(t < NCHUNK // 2 - 1)
    def _():
      pltpu.async_copy(dst_hbm.at[pl.ds(r0 + 2 * CHUNK_E, CHUNK_E)], didxa,
                       sema)

    pltpu.make_async_copy(dst_hbm.at[pl.ds(r0 + CHUNK_E, CHUNK_E)], didxb,
                          semb).wait()
    process(didxb)
    return carry

  lax.fori_loop(0, NCHUNK // 2, body2, 0)
  pltpu.sync_copy(dslab, out_hbm.at[tid])


@functools.partial(
    pl.kernel,
    out_type=jax.ShapeDtypeStruct((2, H, NP), jnp.float32),
    mesh=_mesh(),
    compiler_params=_sc_reg_params,
    scratch_types=[
        pltpu.VMEM((FB, NP), jnp.float32),
        pltpu.VMEM((FB, NP), jnp.float32),
        pltpu.VMEM((CHUNK_E,), jnp.int32),
        pltpu.VMEM((CHUNK_E,), jnp.int32),
        pltpu.VMEM((CHUNK_E,), jnp.int32),
        pltpu.VMEM((CHUNK_E,), jnp.int32),
        pltpu.SemaphoreType.DMA,
        pltpu.SemaphoreType.DMA,
    ],
)
def _sc_layer_reg(y_hbm, src_hbm, dst_hbm, out_hbm, yslab, zslab,
                  sidxa, didxa, sidxb, didxb, sema, semb):
  cid = lax.axis_index("c")
  sid = lax.axis_index("s")
  pltpu.async_copy(y_hbm.at[pl.ds(sid * FB, FB)], yslab, semb)
  zero16 = jnp.zeros((16,), jnp.float32)

  @plsc.parallel_loop(0, NP // 16, unroll=8)
  def _zero(i):
    for f in range(FB):
      zslab[f, pl.ds(i * 16, 16)] = zero16

  pltpu.make_async_copy(y_hbm.at[pl.ds(sid * FB, FB)], yslab, semb).wait()

  fsplat = [jnp.full((16,), f, jnp.int32) for f in range(FB)]

  def process(sidx, didx):
    @plsc.parallel_loop(0, CHUNK_E // 16, unroll=4)
    def _groups(i):
      s = sidx[pl.ds(i * 16, 16)]
      d = didx[pl.ds(i * 16, 16)]
      for f in range(FB):
        v = plsc.load_gather(yslab, [fsplat[f], s])
        plsc.addupdate_scatter(zslab, [fsplat[f], d], v)

  ndbl = jnp.where(cid == 0, NDBL0, NDBL_TOTAL - NDBL0)
  ebase = jnp.where(cid == 0, 0, NDBL0 * 2 * CHUNK_E)

  def wait_chunk(r, sidx, didx, sem):
    pltpu.make_async_copy(src_hbm.at[pl.ds(ebase + r, CHUNK_E)], sidx,
                          sem).wait()
    pltpu.make_async_copy(dst_hbm.at[pl.ds(ebase + r, CHUNK_E)], didx,
                          sem).wait()

  def load_chunk(r, sidx, didx, sem):
    pltpu.async_copy(src_hbm.at[pl.ds(ebase + r, CHUNK_E)], sidx, sem)
    pltpu.async_copy(dst_hbm.at[pl.ds(ebase + r, CHUNK_E)], didx, sem)

  load_chunk(0, sidxa, didxa, sema)

  def body2(t, carry):
    r0 = t * 2 * CHUNK_E
    load_chunk(r0 + CHUNK_E, sidxb, didxb, semb)
    wait_chunk(r0, sidxa, didxa, sema)
    process(sidxa, didxa)

    @pl.when(t < ndbl - 1)
    def _():
      load_chunk(r0 + 2 * CHUNK_E, sidxa, didxa, sema)

    wait_chunk(r0 + CHUNK_E, sidxb, didxb, semb)
    process(sidxb, didxb)
    return carry

  lax.fori_loop(0, ndbl, body2, 0)
  pltpu.sync_copy(zslab, out_hbm.at[cid, pl.ds(sid * FB, FB)])


def _tc_pre_body(deg_ref, xt_ref, w1t_ref, dinv_ref, y1_ref):
  deg = jnp.sum(deg_ref[...], axis=0, keepdims=True) + 1.0
  dinv = lax.rsqrt(deg)
  dinv_ref[...] = dinv
  xw = jnp.dot(w1t_ref[...], xt_ref[...], preferred_element_type=jnp.float32)
  y1_ref[...] = xw * dinv


def _tc_pre(deg2, x_fm, w1t):
  return pl.pallas_call(
      _tc_pre_body,
      out_shape=(jax.ShapeDtypeStruct((1, NP), jnp.float32),
                 jax.ShapeDtypeStruct((H, NP), jnp.float32)),
  )(deg2, x_fm, w1t)


def _tc_mid_body(z_ref, y_ref, dinv_ref, b_ref, wt_ref, ynext_ref):
  z = z_ref[0] + z_ref[1] + y_ref[...]
  dinv = dinv_ref[...]
  h = jnp.maximum(z * dinv + b_ref[...], 0.0)
  ynext_ref[...] = jnp.dot(
      wt_ref[...], h, preferred_element_type=jnp.float32) * dinv


def _tc_mid(z_parts, y_prev, dinv, b_col, wt_next):
  return pl.pallas_call(
      _tc_mid_body,
      out_shape=jax.ShapeDtypeStruct((H, NP), jnp.float32),
  )(z_parts, y_prev, dinv, b_col, wt_next)


def _tc_final_body(z_ref, y_ref, dinv_ref, b_ref, batch_ref, wl_ref, bl_ref,
                   out_ref, hpool_ref):
  z = z_ref[0] + z_ref[1] + y_ref[...]
  h = z * dinv_ref[...] + b_ref[...]
  h_n = h[:, :N]
  gids = lax.broadcasted_iota(jnp.int32, (N, G), 1)
  seg = (batch_ref[...] == gids).astype(jnp.float32)
  sums = jnp.dot(h_n, seg, preferred_element_type=jnp.float32)
  cnt = jnp.sum(seg, axis=0, keepdims=True)
  hpool_fm = sums / jnp.maximum(cnt, 1.0)
  eye = (lax.broadcasted_iota(jnp.int32, (H, H), 0) ==
         lax.broadcasted_iota(jnp.int32, (H, H), 1)).astype(jnp.float32)
  hpool_ref[...] = lax.dot_general(
      hpool_fm, eye, (((0,), (0,)), ((), ())),
      preferred_element_type=jnp.float32)
  out_ref[...] = lax.dot_general(
      hpool_fm, wl_ref[...], (((0,), (0,)), ((), ())),
      preferred_element_type=jnp.float32) + bl_ref[...]


def _tc_final(z_parts, y3, dinv, b3_col, batch_col, wl, bl):
  return pl.pallas_call(
      _tc_final_body,
      out_shape=(jax.ShapeDtypeStruct((G, C), jnp.float32),
                 jax.ShapeDtypeStruct((G, H), jnp.float32)),
  )(z_parts, y3, dinv, b3_col, batch_col, wl, bl)


def kernel(x, edge_index, batch, W1, b1, W2, b2, W3, b3, Wl, bl):
  src = edge_index[0]
  dst = edge_index[1]
  pad = jnp.full((EP - E,), N, dtype=jnp.int32)
  src1 = jnp.concatenate([src, pad])
  dst1 = jnp.concatenate([dst, pad])
  x_fm = jnp.pad(jnp.transpose(x), ((0, 0), (0, NP - N)))
  batch_col = batch.reshape(N, 1)

  deg_parts = _sc_degree(dst1)
  dinv, y1 = _tc_pre(deg_parts, x_fm, jnp.transpose(W1))

  z1 = _sc_layer_reg(y1, src1, dst1)
  y2 = _tc_mid(z1, y1, dinv, b1.reshape(H, 1), jnp.transpose(W2))

  z2 = _sc_layer_reg(y2, src1, dst1)
  y3 = _tc_mid(z2, y2, dinv, b2.reshape(H, 1), jnp.transpose(W3))

  z3 = _sc_layer_reg(y3, src1, dst1)
  out, hpool = _tc_final(z3, y3, dinv, b3.reshape(H, 1), batch_col,
                         Wl, bl.reshape(1, C))
  return (out, hpool)


# speedup vs baseline: 1.1599x; 1.0001x over previous
import functools

import jax
import jax.numpy as jnp
from jax import lax
from jax.experimental import pallas as pl
from jax.experimental.pallas import tpu as pltpu
from jax.experimental.pallas import tpu_sc as plsc

N = 10000
E = 320000
F_IN = 128
H = 64
C = 10
G = 64

NP = 10240
EP = 327680
EDGE_ROWS = EP // 128
TILE_EDGE_ROWS = EDGE_ROWS // 32
CHUNK_E = 1024
NCHUNK = TILE_EDGE_ROWS * 128 // CHUNK_E
NDBL_TOTAL = EP // (2 * CHUNK_E)
NDBL0 = 90
FB = 4
FG = H // FB

_mesh = functools.partial(
    plsc.VectorSubcoreMesh, core_axis_name="c", subcore_axis_name="s")
_sc_reg_params = pltpu.CompilerParams(
    use_tc_tiling_on_sc=False, needs_layout_passes=False)


@functools.partial(
    pl.kernel,
    out_type=jax.ShapeDtypeStruct((32, NP), jnp.float32),
    mesh=_mesh(),
    compiler_params=_sc_reg_params,
    scratch_types=[
        pltpu.VMEM((NP,), jnp.float32),
        pltpu.VMEM((CHUNK_E,), jnp.int32),
        pltpu.VMEM((CHUNK_E,), jnp.int32),
        pltpu.SemaphoreType.DMA,
        pltpu.SemaphoreType.DMA,
    ],
)
def _sc_degree(dst_hbm, out_hbm, dslab, didxa, didxb, sema, semb):
  cid = lax.axis_index("c")
  sid = lax.axis_index("s")
  tid = cid * 16 + sid
  base = tid * TILE_EDGE_ROWS * 128
  pltpu.async_copy(dst_hbm.at[pl.ds(base, CHUNK_E)], didxa, sema)
  zero16 = jnp.zeros((16,), jnp.float32)

  @plsc.parallel_loop(0, NP // 16, unroll=8)
  def _zero(i):
    dslab[pl.ds(i * 16, 16)] = zero16

  ones16 = jnp.ones((16,), jnp.float32)

  def process(didx):
    @plsc.parallel_loop(0, CHUNK_E // 16, unroll=8)
    def _groups(i):
      d = didx[pl.ds(i * 16, 16)]
      plsc.addupdate_scatter(dslab, [d], ones16)

  def body2(t, carry):
    r0 = base + t * 2 * CHUNK_E
    pltpu.async_copy(dst_hbm.at[pl.ds(r0 + CHUNK_E, CHUNK_E)], didxb, semb)
    pltpu.make_async_copy(dst_hbm.at[pl.ds(r0, CHUNK_E)], didxa, sema).wait()
    process(didxa)

    @pl.when(t < NCHUNK // 2 - 1)
    def _():
      pltpu.async_copy(dst_hbm.at[pl.ds(r0 + 2 * CHUNK_E, CHUNK_E)], didxa,
                       sema)

    pltpu.make_async_copy(dst_hbm.at[pl.ds(r0 + CHUNK_E, CHUNK_E)], didxb,
                          semb).wait()
    process(didxb)
    return carry

  lax.fori_loop(0, NCHUNK // 2, body2, 0)
  pltpu.sync_copy(dslab, out_hbm.at[tid])


@functools.partial(
    pl.kernel,
    out_type=jax.ShapeDtypeStruct((2, H, NP), jnp.float32),
    mesh=_mesh(),
    compiler_params=_sc_reg_params,
    scratch_types=[
        pltpu.VMEM((FB, NP), jnp.float32),
        pltpu.VMEM((FB, NP), jnp.float32),
        pltpu.VMEM((CHUNK_E,), jnp.int32),
        pltpu.VMEM((CHUNK_E,), jnp.int32),
        pltpu.VMEM((CHUNK_E,), jnp.int32),
        pltpu.VMEM((CHUNK_E,), jnp.int32),
        pltpu.SemaphoreType.DMA,
        pltpu.SemaphoreType.DMA,
    ],
)
def _sc_layer_reg(y_hbm, src_hbm, dst_hbm, out_hbm, yslab, zslab,
                  sidxa, didxa, sidxb, didxb, sema, semb):
  cid = lax.axis_index("c")
  sid = lax.axis_index("s")
  pltpu.async_copy(y_hbm.at[pl.ds(sid * FB, FB)], yslab, semb)
  zero16 = jnp.zeros((16,), jnp.float32)

  @plsc.parallel_loop(0, NP // 16, unroll=8)
  def _zero(i):
    for f in range(FB):
      zslab[f, pl.ds(i * 16, 16)] = zero16

  pltpu.make_async_copy(y_hbm.at[pl.ds(sid * FB, FB)], yslab, semb).wait()

  fsplat = [jnp.full((16,), f, jnp.int32) for f in range(FB)]

  def process(sidx, didx):
    @plsc.parallel_loop(0, CHUNK_E // 16, unroll=4)
    def _groups(i):
      s = sidx[pl.ds(i * 16, 16)]
      d = didx[pl.ds(i * 16, 16)]
      for f in range(FB):
        v = plsc.load_gather(yslab, [fsplat[f], s])
        plsc.addupdate_scatter(zslab, [fsplat[f], d], v)

  ndbl = jnp.where(cid == 0, NDBL0, NDBL_TOTAL - NDBL0)
  ebase = jnp.where(cid == 0, 0, NDBL0 * 2 * CHUNK_E)

  def wait_chunk(r, sidx, didx, sem):
    pltpu.make_async_copy(src_hbm.at[pl.ds(ebase + r, CHUNK_E)], sidx,
                          sem).wait()
    pltpu.make_async_copy(dst_hbm.at[pl.ds(ebase + r, CHUNK_E)], didx,
                          sem).wait()

  def load_chunk(r, sidx, didx, sem):
    pltpu.async_copy(src_hbm.at[pl.ds(ebase + r, CHUNK_E)], sidx, sem)
    pltpu.async_copy(dst_hbm.at[pl.ds(ebase + r, CHUNK_E)], didx, sem)

  load_chunk(0, sidxa, didxa, sema)

  def body2(t, carry):
    r0 = t * 2 * CHUNK_E
    load_chunk(r0 + CHUNK_E, sidxb, didxb, semb)
    wait_chunk(r0, sidxa, didxa, sema)
    process(sidxa, didxa)

    @pl.when(t < ndbl - 1)
    def _():
      load_chunk(r0 + 2 * CHUNK_E, sidxa, didxa, sema)

    wait_chunk(r0 + CHUNK_E, sidxb, didxb, semb)
    process(sidxb, didxb)
    return carry

  lax.fori_loop(0, ndbl, body2, 0)
  pltpu.sync_copy(zslab, out_hbm.at[cid, pl.ds(sid * FB, FB)])


def _tc_pre_body(deg_ref, x_ref, w1_ref, dinv_ref, y1_ref):
  deg = jnp.sum(deg_ref[...], axis=0, keepdims=True) + 1.0
  dinv = lax.rsqrt(deg)
  dinv_ref[...] = dinv
  xw = lax.dot_general(w1_ref[...], x_ref[...], (((0,), (1,)), ((), ())),
                       preferred_element_type=jnp.float32)
  y1_ref[...] = xw * dinv


def _tc_pre(deg2, x_p, w1):
  return pl.pallas_call(
      _tc_pre_body,
      out_shape=(jax.ShapeDtypeStruct((1, NP), jnp.float32),
                 jax.ShapeDtypeStruct((H, NP), jnp.float32)),
  )(deg2, x_p, w1)


def _tc_mid_body(z_ref, y_ref, dinv_ref, b_ref, wt_ref, ynext_ref):
  z = z_ref[0] + z_ref[1] + y_ref[...]
  dinv = dinv_ref[...]
  h = jnp.maximum(z * dinv + b_ref[...], 0.0)
  ynext_ref[...] = jnp.dot(
      wt_ref[...], h, preferred_element_type=jnp.float32) * dinv


def _tc_mid(z_parts, y_prev, dinv, b_col, wt_next):
  return pl.pallas_call(
      _tc_mid_body,
      out_shape=jax.ShapeDtypeStruct((H, NP), jnp.float32),
  )(z_parts, y_prev, dinv, b_col, wt_next)


def _tc_final_body(z_ref, y_ref, dinv_ref, b_ref, batch_ref, wl_ref, bl_ref,
                   out_ref, hpool_ref):
  z = z_ref[0] + z_ref[1] + y_ref[...]
  h = z * dinv_ref[...] + b_ref[...]
  h_n = h[:, :N]
  gids = lax.broadcasted_iota(jnp.int32, (N, G), 1)
  seg = (batch_ref[...] == gids).astype(jnp.float32)
  sums = jnp.dot(h_n, seg, preferred_element_type=jnp.float32)
  cnt = jnp.sum(seg, axis=0, keepdims=True)
  hpool_fm = sums / jnp.maximum(cnt, 1.0)
  eye = (lax.broadcasted_iota(jnp.int32, (H, H), 0) ==
         lax.broadcasted_iota(jnp.int32, (H, H), 1)).astype(jnp.float32)
  hpool_ref[...] = lax.dot_general(
      hpool_fm, eye, (((0,), (0,)), ((), ())),
      preferred_element_type=jnp.float32)
  out_ref[...] = lax.dot_general(
      hpool_fm, wl_ref[...], (((0,), (0,)), ((), ())),
      preferred_element_type=jnp.float32) + bl_ref[...]


def _tc_final(z_parts, y3, dinv, b3_col, batch_col, wl, bl):
  return pl.pallas_call(
      _tc_final_body,
      out_shape=(jax.ShapeDtypeStruct((G, C), jnp.float32),
                 jax.ShapeDtypeStruct((G, H), jnp.float32)),
  )(z_parts, y3, dinv, b3_col, batch_col, wl, bl)


def kernel(x, edge_index, batch, W1, b1, W2, b2, W3, b3, Wl, bl):
  src = edge_index[0]
  dst = edge_index[1]
  pad = jnp.full((EP - E,), N, dtype=jnp.int32)
  src1 = jnp.concatenate([src, pad])
  dst1 = jnp.concatenate([dst, pad])
  x_p = jnp.pad(x, ((0, NP - N), (0, 0)))
  batch_col = batch.reshape(N, 1)

  deg_parts = _sc_degree(dst1)
  dinv, y1 = _tc_pre(deg_parts, x_p, W1)

  z1 = _sc_layer_reg(y1, src1, dst1)
  y2 = _tc_mid(z1, y1, dinv, b1.reshape(H, 1), jnp.transpose(W2))

  z2 = _sc_layer_reg(y2, src1, dst1)
  y3 = _tc_mid(z2, y2, dinv, b2.reshape(H, 1), jnp.transpose(W3))

  z3 = _sc_layer_reg(y3, src1, dst1)
  out, hpool = _tc_final(z3, y3, dinv, b3.reshape(H, 1), batch_col,
                         Wl, bl.reshape(1, C))
  return (out, hpool)
